# baseline (speedup 1.0000x reference)
import functools

import jax
import jax.numpy as jnp
from jax import lax
from jax.experimental import pallas as pl
from jax.experimental.pallas import tpu as pltpu
from jax.experimental.pallas import tpu_sc as plsc

N_ROWS = 1000000
D = 32
B = 16384


BC = 98304
GRID = (N_ROWS + BC - 1) // BC


def _matvec_body(utt_ref, itt_ref, w_ref, b_ref, yu_ref, yi_ref):
    u = utt_ref[...]
    it = itt_ref[...]
    wu = w_ref[0:1, :D]
    wi = w_ref[0:1, D:]
    yu = jax.lax.dot_general(wu, u, (((1,), (0,)), ((), ())),
                             preferred_element_type=jnp.float32)
    yi = jax.lax.dot_general(wi, it, (((1,), (0,)), ((), ())),
                             preferred_element_type=jnp.float32)
    yu_ref[...] = yu[0] + b_ref[0]
    yi_ref[...] = yi[0]


_matvec = pl.pallas_call(
    _matvec_body,
    grid=(GRID,),
    in_specs=[
        pl.BlockSpec((D, BC), lambda i: (0, i)),
        pl.BlockSpec((D, BC), lambda i: (0, i)),
        pl.BlockSpec((1, 2 * D), lambda i: (0, 0)),
        pl.BlockSpec(memory_space=pltpu.SMEM),
    ],
    out_specs=[
        pl.BlockSpec((BC,), lambda i: (i,)),
        pl.BlockSpec((BC,), lambda i: (i,)),
    ],
    out_shape=[
        jax.ShapeDtypeStruct((N_ROWS,), jnp.float32),
        jax.ShapeDtypeStruct((N_ROWS,), jnp.float32),
    ],
)


L = 16
NW = 32
BPW = B // NW
NIDX = 128
NCHUNK = BPW // NIDX

_mesh = plsc.VectorSubcoreMesh(core_axis_name="c", subcore_axis_name="s")


@functools.partial(
    pl.kernel,
    mesh=_mesh,
    out_type=jax.ShapeDtypeStruct((B,), jnp.float32),
    compiler_params=pltpu.CompilerParams(
        needs_layout_passes=False, use_tc_tiling_on_sc=False
    ),
    scratch_types=[
        pltpu.VMEM((NCHUNK, NIDX), jnp.int32),
        pltpu.VMEM((NCHUNK, NIDX), jnp.int32),
        pltpu.VMEM((BPW,), jnp.float32),
        pltpu.VMEM((BPW,), jnp.float32),
        pltpu.VMEM((BPW,), jnp.float32),
        pltpu.SemaphoreType.DMA,
    ],
)
def _sc_gather_add(uid_hbm, iid_hbm, yu_hbm, yi_hbm, out_hbm,
                   idx_u, idx_i, vals_u, vals_i, out_v, sem):
    wid = lax.axis_index("s") * 2 + lax.axis_index("c")
    base = wid * BPW

    pltpu.sync_copy(uid_hbm.at[wid], idx_u)
    pltpu.sync_copy(iid_hbm.at[wid], idx_i)

    copies = []
    for k in range(NCHUNK):
        dst = pl.ds(k * NIDX, NIDX)
        copies.append(pltpu.async_copy(yu_hbm.at[idx_u.at[k]], vals_u.at[dst], sem))
        copies.append(pltpu.async_copy(yi_hbm.at[idx_i.at[k]], vals_i.at[dst], sem))
    for cp in copies:
        cp.wait()

    for g in range(BPW // L):
        sl = pl.ds(g * L, L)
        out_v[sl] = vals_u[sl] + vals_i[sl]

    pltpu.sync_copy(out_v, out_hbm.at[pl.ds(base, BPW)])


def kernel(user_ids, item_ids, user_table, item_table, W, b):
    utt = user_table.T
    itt = item_table.T
    bs = b.reshape(1).astype(jnp.float32)
    yu, yi = _matvec(utt, itt, W.astype(jnp.float32), bs)

    uid = user_ids.astype(jnp.int32).reshape(NW, NCHUNK, NIDX)
    iid = item_ids.astype(jnp.int32).reshape(NW, NCHUNK, NIDX)
    return _sc_gather_add(uid, iid, yu, yi)


# speedup vs baseline: 1.0478x; 1.0478x over previous
import functools

import jax
import jax.numpy as jnp
from jax import lax
from jax.experimental import pallas as pl
from jax.experimental.pallas import tpu as pltpu
from jax.experimental.pallas import tpu_sc as plsc

N_ROWS = 1000000
D = 32
B = 16384


BC = 100352
GRID = (N_ROWS + BC - 1) // BC


def _matvec_body(utt_ref, itt_ref, w_ref, b_ref, yu_ref, yi_ref):
    u = utt_ref[...]
    it = itt_ref[...]
    wu = w_ref[0:1, :D]
    wi = w_ref[0:1, D:]
    yu = jax.lax.dot_general(wu, u, (((1,), (0,)), ((), ())),
                             preferred_element_type=jnp.float32)
    yi = jax.lax.dot_general(wi, it, (((1,), (0,)), ((), ())),
                             preferred_element_type=jnp.float32)
    yu_ref[...] = yu[0] + b_ref[0]
    yi_ref[...] = yi[0]


_matvec = pl.pallas_call(
    _matvec_body,
    grid=(GRID,),
    in_specs=[
        pl.BlockSpec((D, BC), lambda i: (0, i)),
        pl.BlockSpec((D, BC), lambda i: (0, i)),
        pl.BlockSpec((1, 2 * D), lambda i: (0, 0)),
        pl.BlockSpec(memory_space=pltpu.SMEM),
    ],
    out_specs=[
        pl.BlockSpec((BC,), lambda i: (i,)),
        pl.BlockSpec((BC,), lambda i: (i,)),
    ],
    out_shape=[
        jax.ShapeDtypeStruct((N_ROWS,), jnp.float32),
        jax.ShapeDtypeStruct((N_ROWS,), jnp.float32),
    ],
)


L = 16
NW = 32
BPW = B // NW
NIDX = 128
NCHUNK = BPW // NIDX

_mesh = plsc.VectorSubcoreMesh(core_axis_name="c", subcore_axis_name="s")


@functools.partial(
    pl.kernel,
    mesh=_mesh,
    out_type=jax.ShapeDtypeStruct((B,), jnp.float32),
    compiler_params=pltpu.CompilerParams(
        needs_layout_passes=False, use_tc_tiling_on_sc=False
    ),
    scratch_types=[
        pltpu.VMEM((NCHUNK, NIDX), jnp.int32),
        pltpu.VMEM((NCHUNK, NIDX), jnp.int32),
        pltpu.VMEM((BPW,), jnp.float32),
        pltpu.VMEM((BPW,), jnp.float32),
        pltpu.VMEM((BPW,), jnp.float32),
        pltpu.SemaphoreType.DMA,
    ],
)
def _sc_gather_add(uid_hbm, iid_hbm, yu_hbm, yi_hbm, out_hbm,
                   idx_u, idx_i, vals_u, vals_i, out_v, sem):
    wid = lax.axis_index("s") * 2 + lax.axis_index("c")
    base = wid * BPW

    pltpu.sync_copy(uid_hbm.at[wid], idx_u)
    pltpu.sync_copy(iid_hbm.at[wid], idx_i)

    copies = []
    for k in range(NCHUNK):
        dst = pl.ds(k * NIDX, NIDX)
        copies.append(pltpu.async_copy(yu_hbm.at[idx_u.at[k]], vals_u.at[dst], sem))
        copies.append(pltpu.async_copy(yi_hbm.at[idx_i.at[k]], vals_i.at[dst], sem))
    for cp in copies:
        cp.wait()

    for g in range(BPW // L):
        sl = pl.ds(g * L, L)
        out_v[sl] = vals_u[sl] + vals_i[sl]

    pltpu.sync_copy(out_v, out_hbm.at[pl.ds(base, BPW)])


def kernel(user_ids, item_ids, user_table, item_table, W, b):
    utt = user_table.T
    itt = item_table.T
    bs = b.reshape(1).astype(jnp.float32)
    yu, yi = _matvec(utt, itt, W.astype(jnp.float32), bs)

    uid = user_ids.astype(jnp.int32).reshape(NW, NCHUNK, NIDX)
    iid = item_ids.astype(jnp.int32).reshape(NW, NCHUNK, NIDX)
    return _sc_gather_add(uid, iid, yu, yi)
